# Initial kernel scaffold; baseline (speedup 1.0000x reference)
import functools

import jax
import jax.numpy as jnp
from jax import lax
from jax.experimental import pallas as pl
from jax.experimental.pallas import tpu as pltpu
from jax.experimental.pallas import tpu_sc as plsc

NC = 2
NS = 16
CH = 128

_MESH = plsc.VectorSubcoreMesh(core_axis_name="c", subcore_axis_name="s")


def _pad_up(v, m):
    return (v + m - 1) // m * m


def _sc_degrees(srcr, dstr, n_pad):
    nch = srcr.shape[1]
    zr = n_pad // NS

    @functools.partial(
        pl.kernel,
        out_type=jax.ShapeDtypeStruct((NC, 2, n_pad), jnp.float32),
        mesh=_MESH,
        scratch_types=[
            pltpu.VMEM((nch, CH), jnp.int32),
            pltpu.VMEM((nch, CH), jnp.int32),
            pltpu.VMEM((CH,), jnp.float32),
            pltpu.VMEM((zr,), jnp.float32),
            pltpu.VMEM_SHARED((n_pad,), jnp.float32),
            pltpu.VMEM_SHARED((n_pad,), jnp.float32),
        ],
    )
    def deg_kernel(src_hbm, dst_hbm, out_hbm, idx_s, idx_d, ones, zb, dego, degi):
        cid = lax.axis_index("c")
        sid = lax.axis_index("s")
        w = cid * NS + sid
        pltpu.sync_copy(src_hbm.at[w], idx_s)
        pltpu.sync_copy(dst_hbm.at[w], idx_d)

        def fill(i, _):
            ones[pl.ds(i * 16, 16)] = jnp.ones((16,), jnp.float32)
            return 0

        lax.fori_loop(0, CH // 16, fill, 0)

        def zfill(i, _):
            zb[pl.ds(i * 16, 16)] = jnp.zeros((16,), jnp.float32)
            return 0

        lax.fori_loop(0, zr // 16, zfill, 0)
        pltpu.sync_copy(zb, dego.at[pl.ds(sid * zr, zr)])
        pltpu.sync_copy(zb, degi.at[pl.ds(sid * zr, zr)])
        plsc.subcore_barrier()

        def body(j, _):
            pltpu.sync_copy(ones, dego.at[idx_s.at[j]], add=True)
            pltpu.sync_copy(ones, degi.at[idx_d.at[j]], add=True)
            return 0

        lax.fori_loop(0, nch, body, 0)
        plsc.subcore_barrier()
        sl = pl.ds(sid * zr, zr)
        pltpu.sync_copy(dego.at[sl], out_hbm.at[cid, 0, sl])
        pltpu.sync_copy(degi.at[sl], out_hbm.at[cid, 1, sl])

    return deg_kernel(srcr, dstr)


def _sc_aggregate(h, srcr, dstr, n_pad, feat):
    nch = srcr.shape[1]
    zr = n_pad // NS
    nzb = zr // CH

    @functools.partial(
        pl.kernel,
        out_type=jax.ShapeDtypeStruct((NC, n_pad, feat), jnp.float32),
        mesh=_MESH,
        scratch_types=[
            pltpu.VMEM((nch, CH), jnp.int32),
            pltpu.VMEM((nch, CH), jnp.int32),
            pltpu.VMEM((CH, feat), jnp.float32),
            pltpu.VMEM_SHARED((n_pad, feat), jnp.float32),
            pltpu.SemaphoreType.DMA,
        ],
    )
    def agg_kernel(h_hbm, src_hbm, dst_hbm, out_hbm, idx_s, idx_d, rows, acc, sem):
        cid = lax.axis_index("c")
        sid = lax.axis_index("s")
        w = cid * NS + sid
        pltpu.sync_copy(src_hbm.at[w], idx_s)
        pltpu.sync_copy(dst_hbm.at[w], idx_d)

        def zrow(i, _):
            for cc in range(feat // 16):
                rows[i, pl.ds(cc * 16, 16)] = jnp.zeros((16,), jnp.float32)
            return 0

        lax.fori_loop(0, CH, zrow, 0)
        for b in range(nzb):
            pltpu.sync_copy(rows, acc.at[pl.ds(sid * zr + b * CH, CH)])
        plsc.subcore_barrier()

        def body(j, _):
            pltpu.async_copy(h_hbm.at[idx_s.at[j]], rows, sem).wait()
            pltpu.sync_copy(rows, acc.at[idx_d.at[j]], add=True)
            return 0

        lax.fori_loop(0, nch, body, 0)
        plsc.subcore_barrier()
        sl = pl.ds(sid * zr, zr)
        pltpu.sync_copy(acc.at[sl], out_hbm.at[cid, sl])

    return agg_kernel(h, srcr, dstr)


_BLK = 2000


def _norm(deg2):
    return lax.rsqrt(jnp.maximum(deg2[0, :] + deg2[1, :], 1.0))


def _tc1(x, w1, dego):
    n, f = x.shape

    def body(x_ref, w_ref, dg_ref, h_ref):
        ns = _norm(dg_ref[...])
        h = jnp.dot(x_ref[...], w_ref[...],
                    preferred_element_type=jnp.float32,
                    precision=lax.Precision.HIGHEST)
        h_ref[...] = h * ns[:, None]

    return pl.pallas_call(
        body,
        grid=(n // _BLK,),
        in_specs=[
            pl.BlockSpec((_BLK, f), lambda i: (i, 0)),
            pl.BlockSpec((f, f), lambda i: (0, 0)),
            pl.BlockSpec((2, _BLK), lambda i: (0, i)),
        ],
        out_specs=pl.BlockSpec((_BLK, f), lambda i: (i, 0)),
        out_shape=jax.ShapeDtypeStruct((n, f), jnp.float32),
    )(x, w1, dego)


def _tc2(a0, a1, dego, degi, b1, w2p, n):
    f = a0.shape[1]
    cp = w2p.shape[1]

    def body(a0_ref, a1_ref, dgo_ref, dgi_ref, b1_ref, w2_ref, z_ref):
        nd = _norm(dgi_ref[...])
        ns = _norm(dgo_ref[...])
        h2 = (a0_ref[...] + a1_ref[...]) * nd[:, None] + b1_ref[...]
        h2 = jnp.maximum(h2, 0.0) * ns[:, None]
        z_ref[...] = jnp.dot(h2, w2_ref[...],
                             preferred_element_type=jnp.float32,
                             precision=lax.Precision.HIGHEST)

    return pl.pallas_call(
        body,
        grid=(n // _BLK,),
        in_specs=[
            pl.BlockSpec((_BLK, f), lambda i: (i, 0)),
            pl.BlockSpec((_BLK, f), lambda i: (i, 0)),
            pl.BlockSpec((2, _BLK), lambda i: (0, i)),
            pl.BlockSpec((2, _BLK), lambda i: (0, i)),
            pl.BlockSpec((1, f), lambda i: (0, 0)),
            pl.BlockSpec((f, cp), lambda i: (0, 0)),
        ],
        out_specs=pl.BlockSpec((_BLK, cp), lambda i: (i, 0)),
        out_shape=jax.ShapeDtypeStruct((n, cp), jnp.float32),
    )(a0, a1, dego, degi, b1, w2p)


def _tc3(q0, q1, degi, b2p, n):
    cp = q0.shape[1]

    def body(q0_ref, q1_ref, dgi_ref, b2_ref, o_ref):
        nd = _norm(dgi_ref[...])
        o_ref[...] = (q0_ref[...] + q1_ref[...]) * nd[:, None] + b2_ref[...]

    return pl.pallas_call(
        body,
        grid=(n // _BLK,),
        in_specs=[
            pl.BlockSpec((_BLK, cp), lambda i: (i, 0)),
            pl.BlockSpec((_BLK, cp), lambda i: (i, 0)),
            pl.BlockSpec((2, _BLK), lambda i: (0, i)),
            pl.BlockSpec((1, cp), lambda i: (0, 0)),
        ],
        out_specs=pl.BlockSpec((_BLK, cp), lambda i: (i, 0)),
        out_shape=jax.ShapeDtypeStruct((n, cp), jnp.float32),
    )(q0, q1, degi, b2p)


def kernel(x, edge_index, W1, b1, W2, b2):
    n, f = x.shape
    e = edge_index.shape[1]
    c = W2.shape[1]
    cp = _pad_up(c, 64)
    n_pad = _pad_up(n + 1, NS * CH)

    src = edge_index[0].astype(jnp.int32)
    dst = edge_index[1].astype(jnp.int32)
    e_pad = _pad_up(e, NC * NS * CH)
    src = jnp.concatenate([src, jnp.zeros((e_pad - e,), jnp.int32)])
    dst = jnp.concatenate([dst, jnp.full((e_pad - e,), n, jnp.int32)])
    nch = e_pad // (NC * NS * CH)
    srcr = src.reshape(NC * NS, nch, CH)
    dstr = dst.reshape(NC * NS, nch, CH)

    degp = _sc_degrees(srcr, dstr, n_pad)
    dego = degp[:, 0, :]
    degi = degp[:, 1, :]

    h = _tc1(x, W1, dego)
    agg1 = _sc_aggregate(h, srcr, dstr, n_pad, f)

    w2p = jnp.zeros((f, cp), jnp.float32).at[:, :c].set(W2)
    b2p = jnp.zeros((1, cp), jnp.float32).at[0, :c].set(b2)
    z = _tc2(agg1[0], agg1[1], dego, degi, b1.reshape(1, f), w2p, n)
    agg2 = _sc_aggregate(z, srcr, dstr, n_pad, cp)

    out = _tc3(agg2[0], agg2[1], degi, b2p, n)
    return out[:, :c]


# speedup vs baseline: 3.9850x; 3.9850x over previous
import functools

import jax
import jax.numpy as jnp
from jax import lax
from jax.experimental import pallas as pl
from jax.experimental.pallas import tpu as pltpu
from jax.experimental.pallas import tpu_sc as plsc

NC = 2
NS = 16
CH = 128

@functools.cache
def _mesh():
    return plsc.VectorSubcoreMesh(core_axis_name="c", subcore_axis_name="s",
                                  num_cores=NC, num_subcores=NS)


def _pad_up(v, m):
    return (v + m - 1) // m * m


def _sc_degrees(srcr, dstr, n_pad):
    nch = srcr.shape[1]
    zr = n_pad // NS

    @functools.partial(
        pl.kernel,
        out_type=jax.ShapeDtypeStruct((NC, 2, n_pad), jnp.float32),
        mesh=_mesh(),
        scratch_types=[
            pltpu.VMEM((nch, CH), jnp.int32),
            pltpu.VMEM((nch, CH), jnp.int32),
            pltpu.VMEM((CH,), jnp.float32),
            pltpu.VMEM((zr,), jnp.float32),
            pltpu.VMEM_SHARED((n_pad,), jnp.float32),
            pltpu.VMEM_SHARED((n_pad,), jnp.float32),
        ],
    )
    def deg_kernel(src_hbm, dst_hbm, out_hbm, idx_s, idx_d, ones, zb, dego, degi):
        cid = lax.axis_index("c")
        sid = lax.axis_index("s")
        w = cid * NS + sid
        pltpu.sync_copy(src_hbm.at[w], idx_s)
        pltpu.sync_copy(dst_hbm.at[w], idx_d)

        def fill(i, _):
            ones[pl.ds(i * 16, 16)] = jnp.ones((16,), jnp.float32)
            return 0

        lax.fori_loop(0, CH // 16, fill, 0)

        def zfill(i, _):
            zb[pl.ds(i * 16, 16)] = jnp.zeros((16,), jnp.float32)
            return 0

        lax.fori_loop(0, zr // 16, zfill, 0)
        pltpu.sync_copy(zb, dego.at[pl.ds(sid * zr, zr)])
        pltpu.sync_copy(zb, degi.at[pl.ds(sid * zr, zr)])
        plsc.subcore_barrier()

        def body(j, _):
            pltpu.sync_copy(ones, dego.at[idx_s.at[j]], add=True)
            pltpu.sync_copy(ones, degi.at[idx_d.at[j]], add=True)
            return 0

        lax.fori_loop(0, nch, body, 0)
        plsc.subcore_barrier()
        sl = pl.ds(sid * zr, zr)
        pltpu.sync_copy(dego.at[sl], out_hbm.at[cid, 0, sl])
        pltpu.sync_copy(degi.at[sl], out_hbm.at[cid, 1, sl])

    return deg_kernel(srcr, dstr)


def _sc_aggregate(h, srcr, dstr, n_pad, feat):
    nch = srcr.shape[1]
    zr = n_pad // NS
    nzb = zr // CH

    @functools.partial(
        pl.kernel,
        out_type=jax.ShapeDtypeStruct((NC, n_pad, feat), jnp.float32),
        mesh=_mesh(),
        scratch_types=[
            pltpu.VMEM((nch, CH), jnp.int32),
            pltpu.VMEM((nch, CH), jnp.int32),
            pltpu.VMEM((CH, feat), jnp.float32),
            pltpu.VMEM_SHARED((n_pad, feat), jnp.float32),
            pltpu.SemaphoreType.DMA,
        ],
    )
    def agg_kernel(h_hbm, src_hbm, dst_hbm, out_hbm, idx_s, idx_d, rows, acc, sem):
        cid = lax.axis_index("c")
        sid = lax.axis_index("s")
        w = cid * NS + sid
        pltpu.sync_copy(src_hbm.at[w], idx_s)
        pltpu.sync_copy(dst_hbm.at[w], idx_d)

        def zrow(i, _):
            for cc in range(feat // 16):
                rows[i, pl.ds(cc * 16, 16)] = jnp.zeros((16,), jnp.float32)
            return 0

        lax.fori_loop(0, CH, zrow, 0)
        for b in range(nzb):
            pltpu.sync_copy(rows, acc.at[pl.ds(sid * zr + b * CH, CH)])
        plsc.subcore_barrier()

        def body(j, _):
            pltpu.async_copy(h_hbm.at[idx_s.at[j]], rows, sem).wait()
            pltpu.sync_copy(rows, acc.at[idx_d.at[j]], add=True)
            return 0

        lax.fori_loop(0, nch, body, 0)
        plsc.subcore_barrier()
        sl = pl.ds(sid * zr, zr)
        pltpu.sync_copy(acc.at[sl], out_hbm.at[cid, sl])

    return agg_kernel(h, srcr, dstr)


_BLK = 2000


def _norm(deg2):
    return lax.rsqrt(jnp.maximum(deg2[:, 0] + deg2[:, 1], 1.0))


def _tc1(x, w1, dego):
    n, f = x.shape

    def body(x_ref, w_ref, dg_ref, h_ref):
        ns = _norm(dg_ref[...])
        h = jnp.dot(x_ref[...], w_ref[...],
                    preferred_element_type=jnp.float32,
                    precision=lax.Precision.HIGHEST)
        h_ref[...] = h * ns[:, None]

    return pl.pallas_call(
        body,
        grid=(n // _BLK,),
        in_specs=[
            pl.BlockSpec((_BLK, f), lambda i: (i, 0)),
            pl.BlockSpec((f, f), lambda i: (0, 0)),
            pl.BlockSpec((_BLK, 2), lambda i: (i, 0)),
        ],
        out_specs=pl.BlockSpec((_BLK, f), lambda i: (i, 0)),
        out_shape=jax.ShapeDtypeStruct((n, f), jnp.float32),
    )(x, w1, dego)


def _tc2(a0, a1, dego, degi, b1, w2p, n):
    f = a0.shape[1]
    cp = w2p.shape[1]

    def body(a0_ref, a1_ref, dgo_ref, dgi_ref, b1_ref, w2_ref, z_ref):
        nd = _norm(dgi_ref[...])
        ns = _norm(dgo_ref[...])
        h2 = (a0_ref[...] + a1_ref[...]) * nd[:, None] + b1_ref[...]
        h2 = jnp.maximum(h2, 0.0) * ns[:, None]
        z_ref[...] = jnp.dot(h2, w2_ref[...],
                             preferred_element_type=jnp.float32,
                             precision=lax.Precision.HIGHEST)

    return pl.pallas_call(
        body,
        grid=(n // _BLK,),
        in_specs=[
            pl.BlockSpec((_BLK, f), lambda i: (i, 0)),
            pl.BlockSpec((_BLK, f), lambda i: (i, 0)),
            pl.BlockSpec((_BLK, 2), lambda i: (i, 0)),
            pl.BlockSpec((_BLK, 2), lambda i: (i, 0)),
            pl.BlockSpec((1, f), lambda i: (0, 0)),
            pl.BlockSpec((f, cp), lambda i: (0, 0)),
        ],
        out_specs=pl.BlockSpec((_BLK, cp), lambda i: (i, 0)),
        out_shape=jax.ShapeDtypeStruct((n, cp), jnp.float32),
    )(a0, a1, dego, degi, b1, w2p)


def _tc3(q0, q1, degi, b2p, n):
    cp = q0.shape[1]

    def body(q0_ref, q1_ref, dgi_ref, b2_ref, o_ref):
        nd = _norm(dgi_ref[...])
        o_ref[...] = (q0_ref[...] + q1_ref[...]) * nd[:, None] + b2_ref[...]

    return pl.pallas_call(
        body,
        grid=(n // _BLK,),
        in_specs=[
            pl.BlockSpec((_BLK, cp), lambda i: (i, 0)),
            pl.BlockSpec((_BLK, cp), lambda i: (i, 0)),
            pl.BlockSpec((_BLK, 2), lambda i: (i, 0)),
            pl.BlockSpec((1, cp), lambda i: (0, 0)),
        ],
        out_specs=pl.BlockSpec((_BLK, cp), lambda i: (i, 0)),
        out_shape=jax.ShapeDtypeStruct((n, cp), jnp.float32),
    )(q0, q1, degi, b2p)


def kernel(x, edge_index, W1, b1, W2, b2):
    n, f = x.shape
    e = edge_index.shape[1]
    c = W2.shape[1]
    cp = _pad_up(c, 128)
    n_pad = _pad_up(n + 1, NS * CH)

    src = edge_index[0].astype(jnp.int32)
    dst = edge_index[1].astype(jnp.int32)
    e_pad = _pad_up(e, NC * NS * CH)
    src = jnp.concatenate([src, jnp.zeros((e_pad - e,), jnp.int32)])
    dst = jnp.concatenate([dst, jnp.full((e_pad - e,), n, jnp.int32)])
    nch = e_pad // (NC * NS * CH)
    srcr = src.reshape(NC * NS, nch, CH)
    dstr = dst.reshape(NC * NS, nch, CH)

    degp = _sc_degrees(srcr, dstr, n_pad)
    dego = degp[:, 0, :].T
    degi = degp[:, 1, :].T

    h = _tc1(x, W1, dego)
    agg1 = _sc_aggregate(h, srcr, dstr, n_pad, f)

    w2p = jnp.zeros((f, cp), jnp.float32).at[:, :c].set(W2)
    b2p = jnp.zeros((1, cp), jnp.float32).at[0, :c].set(b2)
    z = _tc2(agg1[0], agg1[1], dego, degi, b1.reshape(1, f), w2p, n)
    agg2 = _sc_aggregate(z, srcr, dstr, n_pad, cp)

    out = _tc3(agg2[0], agg2[1], degi, b2p, n)
    return out[:, :c]
